# baseline (speedup 1.0000x reference)
import functools

import jax
import jax.numpy as jnp
from jax import lax
from jax.experimental import pallas as pl
from jax.experimental.pallas import tpu as pltpu
from jax.experimental.pallas import tpu_sc as plsc

N = 10000
D = 128
NPAD = 10240
CHUNK = 128
NC = 2
NS = 16
ROWS_PER_TILE = NPAD // NS
ROW_CHUNKS = ROWS_PER_TILE // CHUNK
MBLK = 1024
NBLK = NPAD // MBLK
DEGW = 16
PC = 2


def _sc_prop(hs_pad, src2d, dst2d, cpw):
    mesh = plsc.VectorSubcoreMesh(core_axis_name="c", subcore_axis_name="s",
                                  num_cores=PC)

    @functools.partial(
        pl.kernel,
        out_type=jax.ShapeDtypeStruct((PC, NPAD, D), jnp.float32),
        mesh=mesh,
        scratch_types=[
            pltpu.VMEM_SHARED((NPAD, D), jnp.float32),
            pltpu.VMEM((CHUNK,), jnp.int32),
            pltpu.VMEM((CHUNK,), jnp.int32),
            pltpu.VMEM((CHUNK, D), jnp.float32),
        ],
    )
    def kern(hs_hbm, src_hbm, dst_hbm, out_hbm, acc_sh, js0, id0, r0):
        c = lax.axis_index("c")
        s = lax.axis_index("s")
        w = c * NS + s
        ck0 = w * cpw

        zeros16 = jnp.zeros((16,), jnp.float32)

        def zrow(r, _):
            for j in range(D // 16):
                r0[r, pl.ds(j * 16, 16)] = zeros16
            return 0

        lax.fori_loop(0, CHUNK, zrow, 0)
        row0 = s * ROWS_PER_TILE
        for j in range(ROW_CHUNKS):
            pltpu.sync_copy(r0, acc_sh.at[pl.ds(row0 + j * CHUNK, CHUNK)])
        plsc.subcore_barrier()

        def body(k, _):
            pltpu.sync_copy(src_hbm.at[ck0 + k], js0)
            pltpu.sync_copy(dst_hbm.at[ck0 + k], id0)
            pltpu.sync_copy(hs_hbm.at[js0], r0)
            pltpu.sync_copy(r0, acc_sh.at[id0], add=True)
            return 0

        lax.fori_loop(0, cpw, body, 0)
        plsc.subcore_barrier()

        for j in range(ROW_CHUNKS):
            r = row0 + j * CHUNK
            pltpu.sync_copy(acc_sh.at[pl.ds(r, CHUNK)], r0)
            pltpu.sync_copy(r0, out_hbm.at[c, pl.ds(r, CHUNK)])

    return kern(hs_pad, src2d, dst2d)


def _sc_deg(dst2d, cpw):
    mesh = plsc.VectorSubcoreMesh(core_axis_name="c", subcore_axis_name="s")

    @functools.partial(
        pl.kernel,
        out_type=jax.ShapeDtypeStruct((NC, NPAD, DEGW), jnp.float32),
        mesh=mesh,
        scratch_types=[
            pltpu.VMEM_SHARED((NPAD, DEGW), jnp.float32),
            pltpu.VMEM((CHUNK,), jnp.int32),
            pltpu.VMEM((CHUNK,), jnp.int32),
            pltpu.VMEM((CHUNK, DEGW), jnp.float32),
            pltpu.VMEM((ROWS_PER_TILE, DEGW), jnp.float32),
            pltpu.SemaphoreType.DMA,
            pltpu.SemaphoreType.DMA,
        ],
    )
    def kern(dst_hbm, out_hbm, cnt_sh, id0, id1, ones, slab, s0, s1):
        c = lax.axis_index("c")
        s = lax.axis_index("s")
        idx_d = (id0, id1)
        ssem = (s0, s1)
        w = c * NS + s
        ck0 = w * cpw

        ones16 = jnp.ones((16,), jnp.float32)
        zeros16 = jnp.zeros((16,), jnp.float32)

        def frow(r, _):
            ones[r, pl.ds(0, 16)] = ones16
            return 0

        lax.fori_loop(0, CHUNK, frow, 0)

        def zrow(r, _):
            slab[r, pl.ds(0, 16)] = zeros16
            return 0

        lax.fori_loop(0, ROWS_PER_TILE, zrow, 0)
        row0 = s * ROWS_PER_TILE
        pltpu.sync_copy(slab, cnt_sh.at[pl.ds(row0, ROWS_PER_TILE)])
        plsc.subcore_barrier()

        def slot(k, b):
            @pl.when(k >= 2)
            def _():
                pltpu.make_async_copy(ones, cnt_sh.at[idx_d[b]],
                                      ssem[b]).wait()

            pltpu.sync_copy(dst_hbm.at[ck0 + k], idx_d[b])
            pltpu.async_copy(ones, cnt_sh.at[idx_d[b]], ssem[b], add=True)

        def body(g, _):
            for b in range(2):
                slot(g * 2 + b, b)
            return 0

        lax.fori_loop(0, cpw // 2, body, 0)
        for b in range(2):
            pltpu.make_async_copy(ones, cnt_sh.at[idx_d[b]], ssem[b]).wait()
        plsc.subcore_barrier()

        pltpu.sync_copy(cnt_sh.at[pl.ds(row0, ROWS_PER_TILE)], slab)
        pltpu.sync_copy(slab, out_hbm.at[c, pl.ds(row0, ROWS_PER_TILE)])

    return kern(dst2d)


def _tc_mm0(x_pad, W_in):
    def kern(x_ref, w_ref, h_ref):
        h_ref[...] = jnp.dot(x_ref[...], w_ref[...],
                             preferred_element_type=jnp.float32)

    return pl.pallas_call(
        kern,
        grid=(NBLK,),
        in_specs=[
            pl.BlockSpec((MBLK, D), lambda i: (i, 0)),
            pl.BlockSpec((D, D), lambda i: (0, 0)),
        ],
        out_specs=pl.BlockSpec((MBLK, D), lambda i: (i, 0)),
        out_shape=jax.ShapeDtypeStruct((NPAD, D), jnp.float32),
    )(x_pad, W_in)


def _tc_scale(deg_parts, h1):
    def kern(dp_ref, h_ref, dis_ref, hs_ref):
        deg = dp_ref[0, :, 0:1] + dp_ref[1, :, 0:1] + 1.0
        dis = jnp.broadcast_to(lax.rsqrt(deg), (MBLK, D))
        dis_ref[...] = dis
        hs_ref[...] = dis * h_ref[...]

    return pl.pallas_call(
        kern,
        grid=(NBLK,),
        in_specs=[
            pl.BlockSpec((NC, MBLK, DEGW), lambda i: (0, i, 0)),
            pl.BlockSpec((MBLK, D), lambda i: (i, 0)),
        ],
        out_specs=[
            pl.BlockSpec((MBLK, D), lambda i: (i, 0)),
            pl.BlockSpec((MBLK, D), lambda i: (i, 0)),
        ],
        out_shape=[
            jax.ShapeDtypeStruct((NPAD, D), jnp.float32),
            jax.ShapeDtypeStruct((NPAD, D), jnp.float32),
        ],
    )(deg_parts, h1)


def _tc_mid(acc, hs_prev, dis_rep, b_row, W_next):
    def kern(a_ref, hp_ref, dis_ref, b_ref, w_ref, hs_ref):
        i = pl.program_id(0)
        dis = dis_ref[...]
        asum = a_ref[0] if PC == 1 else a_ref[0] + a_ref[1]
        pre = dis * (asum + hp_ref[...]) + b_ref[...]
        gid = i * MBLK + lax.broadcasted_iota(jnp.int32, (MBLK, D), 0)
        xn = jnp.where(gid < N, jnp.maximum(pre, 0.0), 0.0)
        h = jnp.dot(xn, w_ref[...], preferred_element_type=jnp.float32)
        hs_ref[...] = dis * h

    return pl.pallas_call(
        kern,
        grid=(NBLK,),
        in_specs=[
            pl.BlockSpec((PC, MBLK, D), lambda i: (0, i, 0)),
            pl.BlockSpec((MBLK, D), lambda i: (i, 0)),
            pl.BlockSpec((MBLK, D), lambda i: (i, 0)),
            pl.BlockSpec((1, D), lambda i: (0, 0)),
            pl.BlockSpec((D, D), lambda i: (0, 0)),
        ],
        out_specs=pl.BlockSpec((MBLK, D), lambda i: (i, 0)),
        out_shape=jax.ShapeDtypeStruct((NPAD, D), jnp.float32),
    )(acc, hs_prev, dis_rep, b_row, W_next)


def _tc_final(acc, hs3, dis_rep, b_row, Wc_pad, bc_row):
    def kern(a_ref, hp_ref, dis_ref, b_ref, wc_ref, bc_ref,
             ne_ref, ge_ref, lg_ref, ssum):
        i = pl.program_id(0)
        asum = a_ref[0] if PC == 1 else a_ref[0] + a_ref[1]
        pre = dis_ref[...] * (asum + hp_ref[...]) + b_ref[...]
        gid = i * MBLK + lax.broadcasted_iota(jnp.int32, (MBLK, D), 0)
        ne = jnp.where(gid < N, pre, 0.0)
        ne_ref[...] = ne
        csum = jnp.sum(ne, axis=0, keepdims=True)

        @pl.when(i == 0)
        def _():
            ssum[...] = csum

        @pl.when(i > 0)
        def _():
            ssum[...] = ssum[...] + csum

        @pl.when(i == NBLK - 1)
        def _():
            ge = ssum[...] * (1.0 / N)
            ge_ref[...] = ge
            lg_ref[...] = jnp.dot(ge, wc_ref[...],
                                  preferred_element_type=jnp.float32) + bc_ref[...]

    return pl.pallas_call(
        kern,
        grid=(NBLK,),
        in_specs=[
            pl.BlockSpec((PC, MBLK, D), lambda i: (0, i, 0)),
            pl.BlockSpec((MBLK, D), lambda i: (i, 0)),
            pl.BlockSpec((MBLK, D), lambda i: (i, 0)),
            pl.BlockSpec((1, D), lambda i: (0, 0)),
            pl.BlockSpec((D, D), lambda i: (0, 0)),
            pl.BlockSpec((1, D), lambda i: (0, 0)),
        ],
        out_specs=[
            pl.BlockSpec((MBLK, D), lambda i: (i, 0)),
            pl.BlockSpec((1, D), lambda i: (0, 0)),
            pl.BlockSpec((1, D), lambda i: (0, 0)),
        ],
        out_shape=[
            jax.ShapeDtypeStruct((NPAD, D), jnp.float32),
            jax.ShapeDtypeStruct((1, D), jnp.float32),
            jax.ShapeDtypeStruct((1, D), jnp.float32),
        ],
        scratch_shapes=[pltpu.VMEM((1, D), jnp.float32)],
    )(acc, hs3, dis_rep, b_row, Wc_pad, bc_row)


def kernel(x, edge_index, W_in, b_in, W_mid, b_mid, W_out, b_out, Wc, bc):
    E = edge_index.shape[1]
    cpw = -(-E // (PC * NS * CHUNK))
    cpw = -(-cpw // (4 * (NC // PC))) * 4 * (NC // PC)
    e_pad = PC * NS * cpw * CHUNK
    cpw_deg = e_pad // (NC * NS * CHUNK)

    x_pad = jnp.zeros((NPAD, D), jnp.float32).at[:N].set(x)
    sent = jnp.full((e_pad - E,), NPAD - 1, jnp.int32)
    src2d = jnp.concatenate([edge_index[0], sent]).reshape(-1, CHUNK)
    dst2d = jnp.concatenate([edge_index[1], sent]).reshape(-1, CHUNK)
    Wc_pad = jnp.zeros((D, D), jnp.float32).at[:, :2].set(Wc)
    bc_row = jnp.zeros((1, D), jnp.float32).at[0, :2].set(bc)

    deg_parts = _sc_deg(dst2d, cpw_deg)
    h1 = _tc_mm0(x_pad, W_in)
    dis_rep, hs1 = _tc_scale(deg_parts, h1)
    acc1 = _sc_prop(hs1, src2d, dst2d, cpw)
    hs2 = _tc_mid(acc1, hs1, dis_rep, b_in.reshape(1, D), W_mid)
    acc2 = _sc_prop(hs2, src2d, dst2d, cpw)
    hs3 = _tc_mid(acc2, hs2, dis_rep, b_mid.reshape(1, D), W_out)
    acc3 = _sc_prop(hs3, src2d, dst2d, cpw)
    ne_pad, ge_row, lg_row = _tc_final(acc3, hs3, dis_rep,
                                       b_out.reshape(1, D), Wc_pad, bc_row)

    node_embeddings = ne_pad[:N]
    graph_embedding = ge_row[0]
    logits = lg_row[0, :2]
    return (node_embeddings, graph_embedding, logits)


# speedup vs baseline: 1.6370x; 1.6370x over previous
import functools

import jax
import jax.numpy as jnp
from jax import lax
from jax.experimental import pallas as pl
from jax.experimental.pallas import tpu as pltpu
from jax.experimental.pallas import tpu_sc as plsc

N = 10000
D = 128
NPAD = 10240
CHUNK = 128
NC = 2
NS = 16
ROWS_PER_TILE = NPAD // NS
ROW_CHUNKS = ROWS_PER_TILE // CHUNK
MBLK = 1024
NBLK = NPAD // MBLK
DEGW = 16
PC = 2


def _sc_prop(hs_pad, src2d, dst2d, cpw):
    mesh = plsc.VectorSubcoreMesh(core_axis_name="c", subcore_axis_name="s",
                                  num_cores=PC)

    WS = PC * NS

    @functools.partial(
        pl.kernel,
        out_type=jax.ShapeDtypeStruct((PC, NPAD, D), jnp.float32),
        mesh=mesh,
        scratch_types=[
            pltpu.VMEM_SHARED((NPAD, D), jnp.float32),
            pltpu.VMEM((CHUNK,), jnp.int32),
            pltpu.VMEM((CHUNK,), jnp.int32),
            pltpu.VMEM((CHUNK,), jnp.int32),
            pltpu.VMEM((CHUNK,), jnp.int32),
            pltpu.VMEM((CHUNK,), jnp.int32),
            pltpu.VMEM((CHUNK,), jnp.int32),
            pltpu.VMEM((CHUNK,), jnp.int32),
            pltpu.VMEM((CHUNK,), jnp.int32),
            pltpu.VMEM((CHUNK, D), jnp.float32),
            pltpu.VMEM((CHUNK, D), jnp.float32),
            pltpu.SemaphoreType.DMA,
            pltpu.SemaphoreType.DMA,
            pltpu.SemaphoreType.DMA,
            pltpu.SemaphoreType.DMA,
            pltpu.SemaphoreType.DMA,
            pltpu.SemaphoreType.DMA,
            pltpu.SemaphoreType.DMA,
            pltpu.SemaphoreType.DMA,
            pltpu.SemaphoreType.DMA,
            pltpu.SemaphoreType.DMA,
            pltpu.SemaphoreType.DMA,
            pltpu.SemaphoreType.DMA,
        ],
    )
    def kern(hs_hbm, src_hbm, dst_hbm, out_hbm, acc_sh,
             js0, js1, js2, js3, id0, id1, id2, id3, r0, r1,
             g0, g1, s0, s1, i0, i1, i2, i3, d0, d1, d2, d3):
        c = lax.axis_index("c")
        s = lax.axis_index("s")
        rows = (r0, r1)
        idx_s = (js0, js1, js2, js3)
        idx_d = (id0, id1, id2, id3)
        gsem = (g0, g1)
        ssem = (s0, s1)
        isem = (i0, i1, i2, i3)
        dsem = (d0, d1, d2, d3)
        w = c * NS + s

        def ck(k):
            return k * WS + w

        pltpu.sync_copy(src_hbm.at[ck(0)], js0)
        pltpu.sync_copy(src_hbm.at[ck(1)], js1)
        pltpu.async_copy(src_hbm.at[ck(2)], js2, i2)
        pltpu.async_copy(src_hbm.at[ck(3)], js3, i3)
        pltpu.sync_copy(dst_hbm.at[ck(0)], id0)
        pltpu.sync_copy(dst_hbm.at[ck(1)], id1)
        pltpu.async_copy(hs_hbm.at[js0], r0, g0)

        zeros16 = jnp.zeros((16,), jnp.float32)

        def zrow(r, _):
            for j in range(D // 16):
                r1[r, pl.ds(j * 16, 16)] = zeros16
            return 0

        lax.fori_loop(0, CHUNK, zrow, 0)
        row0 = s * ROWS_PER_TILE
        for j in range(ROW_CHUNKS):
            pltpu.sync_copy(r1, acc_sh.at[pl.ds(row0 + j * CHUNK, CHUNK)])
        plsc.subcore_barrier()
        pltpu.async_copy(hs_hbm.at[js1], r1, g1)

        def slot(k, b, dk):
            k2 = k + 2
            dk2 = (dk + 2) % 4
            pltpu.make_async_copy(hs_hbm.at[idx_s[dk]], rows[b],
                                  gsem[b]).wait()

            @pl.when(k + 4 < cpw)
            def _():
                pltpu.async_copy(src_hbm.at[ck(k + 4)], idx_s[dk], isem[dk])

            @pl.when(k >= 2)
            def _():
                pltpu.make_async_copy(dst_hbm.at[ck(k)], idx_d[dk],
                                      dsem[dk]).wait()

            pltpu.async_copy(
                rows[b], acc_sh.at[idx_d[dk]], ssem[b], add=True)

            @pl.when(k2 < cpw)
            def _():
                pltpu.async_copy(dst_hbm.at[ck(k2)], idx_d[dk2], dsem[dk2])

            pltpu.make_async_copy(
                rows[b], acc_sh.at[idx_d[dk]], ssem[b]).wait()

            @pl.when(k2 < cpw)
            def _():
                pltpu.make_async_copy(src_hbm.at[ck(k2)], idx_s[dk2],
                                      isem[dk2]).wait()
                pltpu.async_copy(hs_hbm.at[idx_s[dk2]], rows[b], gsem[b])

        def body(g, _):
            for b in range(4):
                slot(g * 4 + b, b % 2, b)
            return 0

        lax.fori_loop(0, cpw // 4, body, 0)
        plsc.subcore_barrier()

        for j in range(ROW_CHUNKS):
            r = row0 + j * CHUNK
            pltpu.sync_copy(acc_sh.at[pl.ds(r, CHUNK)], r0)
            pltpu.sync_copy(r0, out_hbm.at[c, pl.ds(r, CHUNK)])

    return kern(hs_pad, src2d, dst2d)


def _sc_deg(dst2d, cpw):
    mesh = plsc.VectorSubcoreMesh(core_axis_name="c", subcore_axis_name="s")

    @functools.partial(
        pl.kernel,
        out_type=jax.ShapeDtypeStruct((NC, NPAD, DEGW), jnp.float32),
        mesh=mesh,
        scratch_types=[
            pltpu.VMEM_SHARED((NPAD, DEGW), jnp.float32),
            pltpu.VMEM((CHUNK,), jnp.int32),
            pltpu.VMEM((CHUNK,), jnp.int32),
            pltpu.VMEM((CHUNK, DEGW), jnp.float32),
            pltpu.VMEM((ROWS_PER_TILE, DEGW), jnp.float32),
            pltpu.SemaphoreType.DMA,
            pltpu.SemaphoreType.DMA,
        ],
    )
    def kern(dst_hbm, out_hbm, cnt_sh, id0, id1, ones, slab, s0, s1):
        c = lax.axis_index("c")
        s = lax.axis_index("s")
        idx_d = (id0, id1)
        ssem = (s0, s1)
        w = c * NS + s
        WS = NC * NS

        ones16 = jnp.ones((16,), jnp.float32)
        zeros16 = jnp.zeros((16,), jnp.float32)

        def frow(r, _):
            ones[r, pl.ds(0, 16)] = ones16
            return 0

        lax.fori_loop(0, CHUNK, frow, 0)

        def zrow(r, _):
            slab[r, pl.ds(0, 16)] = zeros16
            return 0

        lax.fori_loop(0, ROWS_PER_TILE, zrow, 0)
        row0 = s * ROWS_PER_TILE
        pltpu.sync_copy(slab, cnt_sh.at[pl.ds(row0, ROWS_PER_TILE)])
        plsc.subcore_barrier()

        def slot(k, b):
            @pl.when(k >= 2)
            def _():
                pltpu.make_async_copy(ones, cnt_sh.at[idx_d[b]],
                                      ssem[b]).wait()

            pltpu.sync_copy(dst_hbm.at[k * WS + w], idx_d[b])
            pltpu.async_copy(ones, cnt_sh.at[idx_d[b]], ssem[b], add=True)

        def body(g, _):
            for b in range(2):
                slot(g * 2 + b, b)
            return 0

        lax.fori_loop(0, cpw // 2, body, 0)
        for b in range(2):
            pltpu.make_async_copy(ones, cnt_sh.at[idx_d[b]], ssem[b]).wait()
        plsc.subcore_barrier()

        pltpu.sync_copy(cnt_sh.at[pl.ds(row0, ROWS_PER_TILE)], slab)
        pltpu.sync_copy(slab, out_hbm.at[c, pl.ds(row0, ROWS_PER_TILE)])

    return kern(dst2d)


def _tc_mm0(x_pad, W_in):
    def kern(x_ref, w_ref, h_ref):
        h_ref[...] = jnp.dot(x_ref[...], w_ref[...],
                             preferred_element_type=jnp.float32)

    return pl.pallas_call(
        kern,
        grid=(NBLK,),
        in_specs=[
            pl.BlockSpec((MBLK, D), lambda i: (i, 0)),
            pl.BlockSpec((D, D), lambda i: (0, 0)),
        ],
        out_specs=pl.BlockSpec((MBLK, D), lambda i: (i, 0)),
        out_shape=jax.ShapeDtypeStruct((NPAD, D), jnp.float32),
    )(x_pad, W_in)


def _tc_scale(deg_parts, h1):
    def kern(dp_ref, h_ref, dis_ref, hs_ref):
        deg = dp_ref[0, :, 0:1] + dp_ref[1, :, 0:1] + 1.0
        dis = jnp.broadcast_to(lax.rsqrt(deg), (MBLK, D))
        dis_ref[...] = dis
        hs_ref[...] = dis * h_ref[...]

    return pl.pallas_call(
        kern,
        grid=(NBLK,),
        in_specs=[
            pl.BlockSpec((NC, MBLK, DEGW), lambda i: (0, i, 0)),
            pl.BlockSpec((MBLK, D), lambda i: (i, 0)),
        ],
        out_specs=[
            pl.BlockSpec((MBLK, D), lambda i: (i, 0)),
            pl.BlockSpec((MBLK, D), lambda i: (i, 0)),
        ],
        out_shape=[
            jax.ShapeDtypeStruct((NPAD, D), jnp.float32),
            jax.ShapeDtypeStruct((NPAD, D), jnp.float32),
        ],
    )(deg_parts, h1)


def _tc_mid(acc, hs_prev, dis_rep, b_row, W_next):
    def kern(a_ref, hp_ref, dis_ref, b_ref, w_ref, hs_ref):
        i = pl.program_id(0)
        dis = dis_ref[...]
        asum = a_ref[0] if PC == 1 else a_ref[0] + a_ref[1]
        pre = dis * (asum + hp_ref[...]) + b_ref[...]
        gid = i * MBLK + lax.broadcasted_iota(jnp.int32, (MBLK, D), 0)
        xn = jnp.where(gid < N, jnp.maximum(pre, 0.0), 0.0)
        h = jnp.dot(xn, w_ref[...], preferred_element_type=jnp.float32)
        hs_ref[...] = dis * h

    return pl.pallas_call(
        kern,
        grid=(NBLK,),
        in_specs=[
            pl.BlockSpec((PC, MBLK, D), lambda i: (0, i, 0)),
            pl.BlockSpec((MBLK, D), lambda i: (i, 0)),
            pl.BlockSpec((MBLK, D), lambda i: (i, 0)),
            pl.BlockSpec((1, D), lambda i: (0, 0)),
            pl.BlockSpec((D, D), lambda i: (0, 0)),
        ],
        out_specs=pl.BlockSpec((MBLK, D), lambda i: (i, 0)),
        out_shape=jax.ShapeDtypeStruct((NPAD, D), jnp.float32),
    )(acc, hs_prev, dis_rep, b_row, W_next)


def _tc_final(acc, hs3, dis_rep, b_row, Wc_pad, bc_row):
    def kern(a_ref, hp_ref, dis_ref, b_ref, wc_ref, bc_ref,
             ne_ref, ge_ref, lg_ref, ssum):
        i = pl.program_id(0)
        asum = a_ref[0] if PC == 1 else a_ref[0] + a_ref[1]
        pre = dis_ref[...] * (asum + hp_ref[...]) + b_ref[...]
        gid = i * MBLK + lax.broadcasted_iota(jnp.int32, (MBLK, D), 0)
        ne = jnp.where(gid < N, pre, 0.0)
        ne_ref[...] = ne
        csum = jnp.sum(ne, axis=0, keepdims=True)

        @pl.when(i == 0)
        def _():
            ssum[...] = csum

        @pl.when(i > 0)
        def _():
            ssum[...] = ssum[...] + csum

        @pl.when(i == NBLK - 1)
        def _():
            ge = ssum[...] * (1.0 / N)
            ge_ref[...] = ge
            lg_ref[...] = jnp.dot(ge, wc_ref[...],
                                  preferred_element_type=jnp.float32) + bc_ref[...]

    return pl.pallas_call(
        kern,
        grid=(NBLK,),
        in_specs=[
            pl.BlockSpec((PC, MBLK, D), lambda i: (0, i, 0)),
            pl.BlockSpec((MBLK, D), lambda i: (i, 0)),
            pl.BlockSpec((MBLK, D), lambda i: (i, 0)),
            pl.BlockSpec((1, D), lambda i: (0, 0)),
            pl.BlockSpec((D, D), lambda i: (0, 0)),
            pl.BlockSpec((1, D), lambda i: (0, 0)),
        ],
        out_specs=[
            pl.BlockSpec((MBLK, D), lambda i: (i, 0)),
            pl.BlockSpec((1, D), lambda i: (0, 0)),
            pl.BlockSpec((1, D), lambda i: (0, 0)),
        ],
        out_shape=[
            jax.ShapeDtypeStruct((NPAD, D), jnp.float32),
            jax.ShapeDtypeStruct((1, D), jnp.float32),
            jax.ShapeDtypeStruct((1, D), jnp.float32),
        ],
        scratch_shapes=[pltpu.VMEM((1, D), jnp.float32)],
    )(acc, hs3, dis_rep, b_row, Wc_pad, bc_row)


def kernel(x, edge_index, W_in, b_in, W_mid, b_mid, W_out, b_out, Wc, bc):
    E = edge_index.shape[1]
    cpw = -(-E // (PC * NS * CHUNK))
    cpw = -(-cpw // (4 * (NC // PC))) * 4 * (NC // PC)
    e_pad = PC * NS * cpw * CHUNK
    cpw_deg = e_pad // (NC * NS * CHUNK)

    x_pad = jnp.zeros((NPAD, D), jnp.float32).at[:N].set(x)
    sent = jnp.full((e_pad - E,), NPAD - 1, jnp.int32)
    src2d = jnp.concatenate([edge_index[0], sent]).reshape(-1, CHUNK)
    dst2d = jnp.concatenate([edge_index[1], sent]).reshape(-1, CHUNK)
    Wc_pad = jnp.zeros((D, D), jnp.float32).at[:, :2].set(Wc)
    bc_row = jnp.zeros((1, D), jnp.float32).at[0, :2].set(bc)

    deg_parts = _sc_deg(dst2d, cpw_deg)
    h1 = _tc_mm0(x_pad, W_in)
    dis_rep, hs1 = _tc_scale(deg_parts, h1)
    acc1 = _sc_prop(hs1, src2d, dst2d, cpw)
    hs2 = _tc_mid(acc1, hs1, dis_rep, b_in.reshape(1, D), W_mid)
    acc2 = _sc_prop(hs2, src2d, dst2d, cpw)
    hs3 = _tc_mid(acc2, hs2, dis_rep, b_mid.reshape(1, D), W_out)
    acc3 = _sc_prop(hs3, src2d, dst2d, cpw)
    ne_pad, ge_row, lg_row = _tc_final(acc3, hs3, dis_rep,
                                       b_out.reshape(1, D), Wc_pad, bc_row)

    node_embeddings = ne_pad[:N]
    graph_embedding = ge_row[0]
    logits = lg_row[0, :2]
    return (node_embeddings, graph_embedding, logits)


# speedup vs baseline: 1.6388x; 1.0011x over previous
import functools

import jax
import jax.numpy as jnp
from jax import lax
from jax.experimental import pallas as pl
from jax.experimental.pallas import tpu as pltpu
from jax.experimental.pallas import tpu_sc as plsc

N = 10000
D = 128
NPAD = 10240
CHUNK = 128
NC = 2
NS = 16
ROWS_PER_TILE = NPAD // NS
ROW_CHUNKS = ROWS_PER_TILE // CHUNK
MBLK = 1024
NBLK = NPAD // MBLK
DEGW = 16
PC = 2


def _sc_prop(hs_pad, src2d, dst2d, cpw):
    mesh = plsc.VectorSubcoreMesh(core_axis_name="c", subcore_axis_name="s",
                                  num_cores=PC)

    WS = PC * NS

    @functools.partial(
        pl.kernel,
        out_type=jax.ShapeDtypeStruct((PC, NPAD, D), jnp.float32),
        mesh=mesh,
        scratch_types=[
            pltpu.VMEM_SHARED((NPAD, D), jnp.float32),
            pltpu.VMEM((CHUNK,), jnp.int32),
            pltpu.VMEM((CHUNK,), jnp.int32),
            pltpu.VMEM((CHUNK,), jnp.int32),
            pltpu.VMEM((CHUNK,), jnp.int32),
            pltpu.VMEM((CHUNK,), jnp.int32),
            pltpu.VMEM((CHUNK,), jnp.int32),
            pltpu.VMEM((CHUNK,), jnp.int32),
            pltpu.VMEM((CHUNK,), jnp.int32),
            pltpu.VMEM((CHUNK, D), jnp.float32),
            pltpu.VMEM((CHUNK, D), jnp.float32),
            pltpu.SemaphoreType.DMA,
            pltpu.SemaphoreType.DMA,
            pltpu.SemaphoreType.DMA,
            pltpu.SemaphoreType.DMA,
            pltpu.SemaphoreType.DMA,
            pltpu.SemaphoreType.DMA,
            pltpu.SemaphoreType.DMA,
            pltpu.SemaphoreType.DMA,
            pltpu.SemaphoreType.DMA,
            pltpu.SemaphoreType.DMA,
            pltpu.SemaphoreType.DMA,
            pltpu.SemaphoreType.DMA,
            pltpu.SemaphoreType.DMA,
        ],
    )
    def kern(hs_hbm, src_hbm, dst_hbm, out_hbm, acc_sh,
             js0, js1, js2, js3, id0, id1, id2, id3, r0, r1,
             g0, g1, s0, s1, i0, i1, i2, i3, d0, d1, d2, d3, zs):
        c = lax.axis_index("c")
        s = lax.axis_index("s")
        rows = (r0, r1)
        idx_s = (js0, js1, js2, js3)
        idx_d = (id0, id1, id2, id3)
        gsem = (g0, g1)
        ssem = (s0, s1)
        isem = (i0, i1, i2, i3)
        dsem = (d0, d1, d2, d3)
        w = c * NS + s

        def ck(k):
            return k * WS + w

        pltpu.sync_copy(src_hbm.at[ck(0)], js0)
        pltpu.sync_copy(src_hbm.at[ck(1)], js1)
        pltpu.async_copy(src_hbm.at[ck(2)], js2, i2)
        pltpu.async_copy(src_hbm.at[ck(3)], js3, i3)
        pltpu.sync_copy(dst_hbm.at[ck(0)], id0)
        pltpu.sync_copy(dst_hbm.at[ck(1)], id1)
        pltpu.async_copy(hs_hbm.at[js0], r0, g0)

        zeros16 = jnp.zeros((16,), jnp.float32)

        def zrow(r, _):
            for j in range(D // 16):
                r1[r, pl.ds(j * 16, 16)] = zeros16
            return 0

        lax.fori_loop(0, CHUNK, zrow, 0)
        row0 = s * ROWS_PER_TILE
        for j in range(ROW_CHUNKS):
            pltpu.async_copy(r1, acc_sh.at[pl.ds(row0 + j * CHUNK, CHUNK)],
                             zs)
        for j in range(ROW_CHUNKS):
            pltpu.make_async_copy(r1, acc_sh.at[pl.ds(row0, CHUNK)],
                                  zs).wait()
        plsc.subcore_barrier()
        pltpu.async_copy(hs_hbm.at[js1], r1, g1)

        def slot(k, b, dk):
            k2 = k + 2
            dk2 = (dk + 2) % 4
            pltpu.make_async_copy(hs_hbm.at[idx_s[dk]], rows[b],
                                  gsem[b]).wait()

            @pl.when(k + 4 < cpw)
            def _():
                pltpu.async_copy(src_hbm.at[ck(k + 4)], idx_s[dk], isem[dk])

            @pl.when(k >= 2)
            def _():
                pltpu.make_async_copy(dst_hbm.at[ck(k)], idx_d[dk],
                                      dsem[dk]).wait()

            pltpu.async_copy(
                rows[b], acc_sh.at[idx_d[dk]], ssem[b], add=True)

            @pl.when(k2 < cpw)
            def _():
                pltpu.async_copy(dst_hbm.at[ck(k2)], idx_d[dk2], dsem[dk2])

            pltpu.make_async_copy(
                rows[b], acc_sh.at[idx_d[dk]], ssem[b]).wait()

            @pl.when(k2 < cpw)
            def _():
                pltpu.make_async_copy(src_hbm.at[ck(k2)], idx_s[dk2],
                                      isem[dk2]).wait()
                pltpu.async_copy(hs_hbm.at[idx_s[dk2]], rows[b], gsem[b])

        def body(g, _):
            for b in range(4):
                slot(g * 4 + b, b % 2, b)
            return 0

        lax.fori_loop(0, cpw // 4, body, 0)
        plsc.subcore_barrier()

        for j in range(ROW_CHUNKS):
            r = row0 + j * CHUNK
            pltpu.async_copy(acc_sh.at[pl.ds(r, CHUNK)],
                             out_hbm.at[c, pl.ds(r, CHUNK)], zs)
        for j in range(ROW_CHUNKS):
            pltpu.make_async_copy(acc_sh.at[pl.ds(row0, CHUNK)],
                                  out_hbm.at[c, pl.ds(row0, CHUNK)],
                                  zs).wait()

    return kern(hs_pad, src2d, dst2d)


def _sc_deg(dst2d, cpw):
    mesh = plsc.VectorSubcoreMesh(core_axis_name="c", subcore_axis_name="s")

    @functools.partial(
        pl.kernel,
        out_type=jax.ShapeDtypeStruct((NC, NPAD, DEGW), jnp.float32),
        mesh=mesh,
        scratch_types=[
            pltpu.VMEM_SHARED((NPAD, DEGW), jnp.float32),
            pltpu.VMEM((CHUNK,), jnp.int32),
            pltpu.VMEM((CHUNK,), jnp.int32),
            pltpu.VMEM((CHUNK, DEGW), jnp.float32),
            pltpu.VMEM((ROWS_PER_TILE, DEGW), jnp.float32),
            pltpu.SemaphoreType.DMA,
            pltpu.SemaphoreType.DMA,
        ],
    )
    def kern(dst_hbm, out_hbm, cnt_sh, id0, id1, ones, slab, s0, s1):
        c = lax.axis_index("c")
        s = lax.axis_index("s")
        idx_d = (id0, id1)
        ssem = (s0, s1)
        w = c * NS + s
        WS = NC * NS

        ones16 = jnp.ones((16,), jnp.float32)
        zeros16 = jnp.zeros((16,), jnp.float32)

        def frow(r, _):
            ones[r, pl.ds(0, 16)] = ones16
            return 0

        lax.fori_loop(0, CHUNK, frow, 0)

        def zrow(r, _):
            slab[r, pl.ds(0, 16)] = zeros16
            return 0

        lax.fori_loop(0, ROWS_PER_TILE, zrow, 0)
        row0 = s * ROWS_PER_TILE
        pltpu.sync_copy(slab, cnt_sh.at[pl.ds(row0, ROWS_PER_TILE)])
        plsc.subcore_barrier()

        def slot(k, b):
            @pl.when(k >= 2)
            def _():
                pltpu.make_async_copy(ones, cnt_sh.at[idx_d[b]],
                                      ssem[b]).wait()

            pltpu.sync_copy(dst_hbm.at[k * WS + w], idx_d[b])
            pltpu.async_copy(ones, cnt_sh.at[idx_d[b]], ssem[b], add=True)

        def body(g, _):
            for b in range(2):
                slot(g * 2 + b, b)
            return 0

        lax.fori_loop(0, cpw // 2, body, 0)
        for b in range(2):
            pltpu.make_async_copy(ones, cnt_sh.at[idx_d[b]], ssem[b]).wait()
        plsc.subcore_barrier()

        pltpu.sync_copy(cnt_sh.at[pl.ds(row0, ROWS_PER_TILE)], slab)
        pltpu.sync_copy(slab, out_hbm.at[c, pl.ds(row0, ROWS_PER_TILE)])

    return kern(dst2d)


def _tc_mm0(x_pad, W_in):
    def kern(x_ref, w_ref, h_ref):
        h_ref[...] = jnp.dot(x_ref[...], w_ref[...],
                             preferred_element_type=jnp.float32)

    return pl.pallas_call(
        kern,
        grid=(NBLK,),
        in_specs=[
            pl.BlockSpec((MBLK, D), lambda i: (i, 0)),
            pl.BlockSpec((D, D), lambda i: (0, 0)),
        ],
        out_specs=pl.BlockSpec((MBLK, D), lambda i: (i, 0)),
        out_shape=jax.ShapeDtypeStruct((NPAD, D), jnp.float32),
    )(x_pad, W_in)


def _tc_scale(deg_parts, h1):
    def kern(dp_ref, h_ref, dis_ref, hs_ref):
        deg = dp_ref[0, :, 0:1] + dp_ref[1, :, 0:1] + 1.0
        dis = jnp.broadcast_to(lax.rsqrt(deg), (MBLK, D))
        dis_ref[...] = dis
        hs_ref[...] = dis * h_ref[...]

    return pl.pallas_call(
        kern,
        grid=(NBLK,),
        in_specs=[
            pl.BlockSpec((NC, MBLK, DEGW), lambda i: (0, i, 0)),
            pl.BlockSpec((MBLK, D), lambda i: (i, 0)),
        ],
        out_specs=[
            pl.BlockSpec((MBLK, D), lambda i: (i, 0)),
            pl.BlockSpec((MBLK, D), lambda i: (i, 0)),
        ],
        out_shape=[
            jax.ShapeDtypeStruct((NPAD, D), jnp.float32),
            jax.ShapeDtypeStruct((NPAD, D), jnp.float32),
        ],
    )(deg_parts, h1)


def _tc_mid(acc, hs_prev, dis_rep, b_row, W_next):
    def kern(a_ref, hp_ref, dis_ref, b_ref, w_ref, hs_ref):
        i = pl.program_id(0)
        dis = dis_ref[...]
        asum = a_ref[0] if PC == 1 else a_ref[0] + a_ref[1]
        pre = dis * (asum + hp_ref[...]) + b_ref[...]
        gid = i * MBLK + lax.broadcasted_iota(jnp.int32, (MBLK, D), 0)
        xn = jnp.where(gid < N, jnp.maximum(pre, 0.0), 0.0)
        h = jnp.dot(xn, w_ref[...], preferred_element_type=jnp.float32)
        hs_ref[...] = dis * h

    return pl.pallas_call(
        kern,
        grid=(NBLK,),
        in_specs=[
            pl.BlockSpec((PC, MBLK, D), lambda i: (0, i, 0)),
            pl.BlockSpec((MBLK, D), lambda i: (i, 0)),
            pl.BlockSpec((MBLK, D), lambda i: (i, 0)),
            pl.BlockSpec((1, D), lambda i: (0, 0)),
            pl.BlockSpec((D, D), lambda i: (0, 0)),
        ],
        out_specs=pl.BlockSpec((MBLK, D), lambda i: (i, 0)),
        out_shape=jax.ShapeDtypeStruct((NPAD, D), jnp.float32),
    )(acc, hs_prev, dis_rep, b_row, W_next)


def _tc_final(acc, hs3, dis_rep, b_row, Wc_pad, bc_row):
    def kern(a_ref, hp_ref, dis_ref, b_ref, wc_ref, bc_ref,
             ne_ref, ge_ref, lg_ref, ssum):
        i = pl.program_id(0)
        asum = a_ref[0] if PC == 1 else a_ref[0] + a_ref[1]
        pre = dis_ref[...] * (asum + hp_ref[...]) + b_ref[...]
        gid = i * MBLK + lax.broadcasted_iota(jnp.int32, (MBLK, D), 0)
        ne = jnp.where(gid < N, pre, 0.0)
        ne_ref[...] = ne
        csum = jnp.sum(ne, axis=0, keepdims=True)

        @pl.when(i == 0)
        def _():
            ssum[...] = csum

        @pl.when(i > 0)
        def _():
            ssum[...] = ssum[...] + csum

        @pl.when(i == NBLK - 1)
        def _():
            ge = ssum[...] * (1.0 / N)
            ge_ref[...] = ge
            lg_ref[...] = jnp.dot(ge, wc_ref[...],
                                  preferred_element_type=jnp.float32) + bc_ref[...]

    return pl.pallas_call(
        kern,
        grid=(NBLK,),
        in_specs=[
            pl.BlockSpec((PC, MBLK, D), lambda i: (0, i, 0)),
            pl.BlockSpec((MBLK, D), lambda i: (i, 0)),
            pl.BlockSpec((MBLK, D), lambda i: (i, 0)),
            pl.BlockSpec((1, D), lambda i: (0, 0)),
            pl.BlockSpec((D, D), lambda i: (0, 0)),
            pl.BlockSpec((1, D), lambda i: (0, 0)),
        ],
        out_specs=[
            pl.BlockSpec((MBLK, D), lambda i: (i, 0)),
            pl.BlockSpec((1, D), lambda i: (0, 0)),
            pl.BlockSpec((1, D), lambda i: (0, 0)),
        ],
        out_shape=[
            jax.ShapeDtypeStruct((NPAD, D), jnp.float32),
            jax.ShapeDtypeStruct((1, D), jnp.float32),
            jax.ShapeDtypeStruct((1, D), jnp.float32),
        ],
        scratch_shapes=[pltpu.VMEM((1, D), jnp.float32)],
    )(acc, hs3, dis_rep, b_row, Wc_pad, bc_row)


def kernel(x, edge_index, W_in, b_in, W_mid, b_mid, W_out, b_out, Wc, bc):
    E = edge_index.shape[1]
    cpw = -(-E // (PC * NS * CHUNK))
    cpw = -(-cpw // 4) * 4
    e_pad = PC * NS * cpw * CHUNK
    cpw_deg = e_pad // (NC * NS * CHUNK)

    x_pad = jnp.zeros((NPAD, D), jnp.float32).at[:N].set(x)
    sent = jnp.full((e_pad - E,), NPAD - 1, jnp.int32)
    src2d = jnp.concatenate([edge_index[0], sent]).reshape(-1, CHUNK)
    dst2d = jnp.concatenate([edge_index[1], sent]).reshape(-1, CHUNK)
    Wc_pad = jnp.zeros((D, D), jnp.float32).at[:, :2].set(Wc)
    bc_row = jnp.zeros((1, D), jnp.float32).at[0, :2].set(bc)

    deg_parts = _sc_deg(dst2d, cpw_deg)
    h1 = _tc_mm0(x_pad, W_in)
    dis_rep, hs1 = _tc_scale(deg_parts, h1)
    acc1 = _sc_prop(hs1, src2d, dst2d, cpw)
    hs2 = _tc_mid(acc1, hs1, dis_rep, b_in.reshape(1, D), W_mid)
    acc2 = _sc_prop(hs2, src2d, dst2d, cpw)
    hs3 = _tc_mid(acc2, hs2, dis_rep, b_mid.reshape(1, D), W_out)
    acc3 = _sc_prop(hs3, src2d, dst2d, cpw)
    ne_pad, ge_row, lg_row = _tc_final(acc3, hs3, dis_rep,
                                       b_out.reshape(1, D), Wc_pad, bc_row)

    node_embeddings = ne_pad[:N]
    graph_embedding = ge_row[0]
    logits = lg_row[0, :2]
    return (node_embeddings, graph_embedding, logits)
